# Initial kernel scaffold; baseline (speedup 1.0000x reference)
import jax
import jax.numpy as jnp
from jax.experimental import pallas as pl


def kernel(x, edge_index, lookup_birth, lookup_gender, symp_tables, Wl1, Wr1, att1, b1, Wl2, Wr2, att2, b2, Wlin, blin):
    raise NotImplementedError("write your pallas kernel here")


# speedup vs baseline: 71.4102x; 71.4102x over previous
import functools

import jax
import jax.numpy as jnp
from jax import lax
from jax.experimental import pallas as pl
from jax.experimental.pallas import tpu as pltpu
from jax.experimental.pallas import tpu_sc as plsc

NC = 2
NS = 16
LN = 16

CH = 128
F = 16


def _dot(a, b):
    return jnp.dot(a, b, precision=lax.Precision.HIGHEST)


def _make_edge_kernel(n_nodes, e_pad, n_heads):
    NW = NC * NS
    per_w = e_pad // NW
    assert per_w % CH == 0
    n_sub = per_w // CH
    n_acc = ((n_nodes + LN + NS * 8 - 1) // (NS * 8)) * (NS * 8)
    hw = F // n_heads
    rows_per_sub = n_acc // NS

    mesh = plsc.VectorSubcoreMesh(
        core_axis_name="c", subcore_axis_name="s", num_cores=NC, num_subcores=NS
    )

    @functools.partial(
        pl.kernel,
        out_type=(
            jax.ShapeDtypeStruct((NC, n_acc, F), jnp.float32),
            jax.ShapeDtypeStruct((NC, n_acc, F), jnp.float32),
        ),
        mesh=mesh,
        compiler_params=pltpu.CompilerParams(needs_layout_passes=False,
                                             use_tc_tiling_on_sc=False),
        scratch_types=[
            pltpu.MemorySpace.VMEM_SHARED((n_acc, F), jnp.float32),
            pltpu.MemorySpace.VMEM_SHARED((n_acc, F), jnp.float32),
            pltpu.VMEM((CH,), jnp.int32),
            pltpu.VMEM((CH,), jnp.int32),
            pltpu.VMEM((CH, F), jnp.float32),
            pltpu.VMEM((CH, F), jnp.float32),
            pltpu.VMEM((CH, F), jnp.float32),
            pltpu.VMEM((CH, F), jnp.float32),
            pltpu.VMEM((F,), jnp.float32),
            pltpu.SemaphoreType.DMA,
            pltpu.SemaphoreType.DMA,
        ],
    )
    def edge_kernel(src2d, dst2d, xl_hbm, xr_hbm, att_hbm, zmsg,
                    msg_out, den_out,
                    acc_msg, acc_den, src_idx, dst_idx, xl_rows, xr_rows,
                    msg_rows, den_rows, att_v, sem_l, sem_r):
        cid = lax.axis_index("c")
        sid = lax.axis_index("s")
        wid = sid * NC + cid

        @pl.when(sid == 0)
        def _():
            pltpu.sync_copy(zmsg, acc_msg)
            pltpu.sync_copy(zmsg, acc_den)

        pltpu.sync_copy(zmsg.at[pl.ds(0, CH)], den_rows)
        pltpu.sync_copy(att_hbm, att_v)
        plsc.subcore_barrier()

        iota = lax.iota(jnp.int32, LN)
        att_arr = att_v[...]
        base_row = wid * n_sub

        def chunk_body(j, carry):
            pltpu.sync_copy(src2d.at[base_row + j], src_idx)
            pltpu.sync_copy(dst2d.at[base_row + j], dst_idx)
            pltpu.async_copy(xl_hbm.at[src_idx], xl_rows, sem_l).wait()
            pltpu.async_copy(xr_hbm.at[dst_idx], xr_rows, sem_r).wait()

            for g in range(CH // LN):
                rows = iota + (g * LN)
                xls = []
                acc = [jnp.zeros((LN,), jnp.float32) for _ in range(n_heads)]
                for c in range(F):
                    colv = jnp.full((LN,), c, jnp.int32)
                    xlc = plsc.load_gather(xl_rows, [rows, colv])
                    xrc = plsc.load_gather(xr_rows, [rows, colv])
                    m = xlc + xrc
                    lk = jnp.maximum(m, m * 0.2)
                    h_i = c // hw
                    acc[h_i] = acc[h_i] + lk * att_arr[c]
                    xls.append(xlc)
                ws = [jnp.exp(a) for a in acc]
                for c in range(F):
                    colv = jnp.full((LN,), c, jnp.int32)
                    plsc.store_scatter(msg_rows, [rows, colv], xls[c] * ws[c // hw])
                for h_i in range(n_heads):
                    colv = jnp.full((LN,), h_i, jnp.int32)
                    plsc.store_scatter(den_rows, [rows, colv], ws[h_i])

            pltpu.sync_copy(msg_rows, acc_msg.at[dst_idx], add=True)
            pltpu.sync_copy(den_rows, acc_den.at[dst_idx], add=True)
            return carry

        lax.fori_loop(0, n_sub, chunk_body, 0)
        plsc.subcore_barrier()

        r0 = sid * rows_per_sub
        pltpu.sync_copy(acc_msg.at[pl.ds(r0, rows_per_sub)],
                        msg_out.at[cid, pl.ds(r0, rows_per_sub)])
        pltpu.sync_copy(acc_den.at[pl.ds(r0, rows_per_sub)],
                        den_out.at[cid, pl.ds(r0, rows_per_sub)])

    return edge_kernel


def _tc_pre(x, A_all, blr, attf):
    n = x.shape[0]
    bn = 2000
    grid = n // bn

    def body(x_ref, a_ref, b_ref, att_ref, xl_ref, xr_ref, sm_ref, sd_ref):
        xx = x_ref[...]
        s = xx[:, 5:20]
        xlr = (_dot(xx[:, 0:5], a_ref[0:5, :]) + _dot(s, a_ref[5:20, :])
               + _dot(s * s, a_ref[20:35, :]) + b_ref[...])
        xl = xlr[:, :F]
        xr = xlr[:, F:]
        m = xl + xr
        lk = jnp.maximum(m, m * 0.2) * att_ref[...]
        a0 = jnp.sum(lk[:, :8], axis=1)
        a1 = jnp.sum(lk[:, 8:], axis=1)
        w0 = jnp.exp(a0)[:, None]
        w1 = jnp.exp(a1)[:, None]
        xl_ref[...] = xl
        xr_ref[...] = xr
        sm_ref[...] = jnp.concatenate([xl[:, :8] * w0, xl[:, 8:] * w1], axis=1)
        sd_ref[...] = jnp.concatenate([w0, w1], axis=1)

    full = lambda shape: pl.BlockSpec(shape, lambda i: (0,) * len(shape))
    return pl.pallas_call(
        body,
        grid=(grid,),
        in_specs=[
            pl.BlockSpec((bn, 20), lambda i: (i, 0)),
            full((35, 2 * F)), full((1, 2 * F)), full((1, F)),
        ],
        out_specs=[
            pl.BlockSpec((bn, F), lambda i: (i, 0)),
            pl.BlockSpec((bn, F), lambda i: (i, 0)),
            pl.BlockSpec((bn, F), lambda i: (i, 0)),
            pl.BlockSpec((bn, 2), lambda i: (i, 0)),
        ],
        out_shape=[
            jax.ShapeDtypeStruct((n, F), jnp.float32),
            jax.ShapeDtypeStruct((n, F), jnp.float32),
            jax.ShapeDtypeStruct((n, F), jnp.float32),
            jax.ShapeDtypeStruct((n, 2), jnp.float32),
        ],
    )(x, A_all, blr, attf)


def _tc_mid(mA, mB, dA, dB, sm1, sd1, b1r, W2cat, att2f):
    n = mA.shape[0]
    bn = 2000
    grid = n // bn

    def body(mA_ref, mB_ref, dA_ref, dB_ref, sm_ref, sd_ref, b_ref, w_ref,
             att_ref, xl_ref, xr_ref, sm2_ref, sd2_ref):
        den = dA_ref[...] + dB_ref[...] + sd_ref[...] + 1e-16
        msg = mA_ref[...] + mB_ref[...] + sm_ref[...]
        out1 = jnp.concatenate(
            [msg[:, :8] / den[:, 0:1], msg[:, 8:] / den[:, 1:2]], axis=1)
        v = out1 + b_ref[...]
        h2 = jnp.where(v > 0, v, jnp.exp(v) - 1.0)
        xlr = _dot(h2, w_ref[...])
        xl = xlr[:, :F]
        xr = xlr[:, F:]
        m = xl + xr
        lk = jnp.maximum(m, m * 0.2) * att_ref[...]
        w = jnp.exp(jnp.sum(lk, axis=1))[:, None]
        xl_ref[...] = xl
        xr_ref[...] = xr
        sm2_ref[...] = xl * w
        sd2_ref[...] = w

    full = lambda shape: pl.BlockSpec(shape, lambda i: (0,) * len(shape))
    bspec = lambda w: pl.BlockSpec((bn, w), lambda i: (i, 0))
    return pl.pallas_call(
        body,
        grid=(grid,),
        in_specs=[bspec(F), bspec(F), bspec(2), bspec(2), bspec(F), bspec(2),
                  full((1, F)), full((F, 2 * F)), full((1, F))],
        out_specs=[bspec(F), bspec(F), bspec(F), bspec(1)],
        out_shape=[
            jax.ShapeDtypeStruct((n, F), jnp.float32),
            jax.ShapeDtypeStruct((n, F), jnp.float32),
            jax.ShapeDtypeStruct((n, F), jnp.float32),
            jax.ShapeDtypeStruct((n, 1), jnp.float32),
        ],
    )(mA, mB, dA, dB, sm1, sd1, b1r, W2cat, att2f)


def _tc_final(mA, mB, dA, dB, sm2, sd2, b2r, Wlin, blin):
    n = mA.shape[0]
    bn = 2000
    grid = n // bn

    def body(mA_ref, mB_ref, dA_ref, dB_ref, sm_ref, sd_ref, b_ref, wl_ref,
             bl_ref, y_ref):
        den = dA_ref[...] + dB_ref[...] + sd_ref[...] + 1e-16
        out2 = (mA_ref[...] + mB_ref[...] + sm_ref[...]) / den
        y_ref[...] = _dot(out2 + b_ref[...], wl_ref[...]) + bl_ref[...]

    full = lambda shape: pl.BlockSpec(shape, lambda i: (0,) * len(shape))
    bspec = lambda w: pl.BlockSpec((bn, w), lambda i: (i, 0))
    return pl.pallas_call(
        body,
        grid=(grid,),
        in_specs=[bspec(F), bspec(F), bspec(1), bspec(1), bspec(F), bspec(1),
                  full((1, F)), full((F, 1)), full((1, 1))],
        out_specs=[bspec(1)],
        out_shape=[jax.ShapeDtypeStruct((n, 1), jnp.float32)],
    )(mA, mB, dA, dB, sm2, sd2, b2r, Wlin, blin)


def kernel(x, edge_index, lookup_birth, lookup_gender, symp_tables,
           Wl1, Wr1, att1, b1, Wl2, Wr2, att2, b2, Wlin, blin):
    n = x.shape[0]
    e = edge_index.shape[1]

    t0 = symp_tables[:, 0]
    t1 = symp_tables[:, 1]
    t2 = symp_tables[:, 2]
    lin = (-1.5 * t0 + 2.0 * t1 - 0.5 * t2) / 15.0
    quad = (0.5 * t0 - t1 + 0.5 * t2) / 15.0
    gd = (lookup_gender[1] - lookup_gender[0])[None, :]
    Wh = jnp.concatenate([lookup_birth, gd, lin, quad], axis=0)
    c0 = lookup_gender[0] + t0.sum(axis=0) / 15.0
    Wcat1 = jnp.concatenate([Wl1, Wr1], axis=1)
    A_all = _dot(Wh, Wcat1) / 3.0
    blr = _dot(c0[None, :], Wcat1) / 3.0
    attf1 = att1.reshape(1, F)
    W2cat = jnp.concatenate([Wl2, Wr2], axis=1)
    att2f = att2.reshape(1, F)
    b1r = b1.reshape(1, F)
    b2r = b2.reshape(1, F)
    blinr = blin.reshape(1, 1)

    src = edge_index[0].astype(jnp.int32)
    dst = edge_index[1].astype(jnp.int32)
    group = NC * NS * CH
    e_pad = ((e + group - 1) // group) * group
    npad = e_pad - e
    if npad:
        pad_ids = jnp.arange(npad, dtype=jnp.int32)
        src = jnp.concatenate([src, pad_ids % n])
        dst = jnp.concatenate([dst, n + (pad_ids % LN)])
    src2d = src.reshape(e_pad // CH, CH)
    dst2d = dst.reshape(e_pad // CH, CH)

    n_acc = ((n + LN + NS * 8 - 1) // (NS * 8)) * (NS * 8)
    zmsg = jnp.zeros((n_acc, F), jnp.float32)

    xl1, xr1, sm1, sd1 = _tc_pre(x, A_all, blr, attf1)
    ek1 = _make_edge_kernel(n, e_pad, 2)
    msg_p, den_p = ek1(src2d, dst2d, xl1, xr1, att1.reshape(F), zmsg)

    xl2, xr2, sm2, sd2 = _tc_mid(msg_p[0, :n], msg_p[1, :n],
                                 den_p[0, :n, :2], den_p[1, :n, :2],
                                 sm1, sd1, b1r, W2cat, att2f)
    ek2 = _make_edge_kernel(n, e_pad, 1)
    msg2_p, den2_p = ek2(src2d, dst2d, xl2, xr2, att2.reshape(F), zmsg)

    return _tc_final(msg2_p[0, :n], msg2_p[1, :n],
                     den2_p[0, :n, :1], den2_p[1, :n, :1],
                     sm2, sd2, b2r, Wlin, blinr)[0]


# speedup vs baseline: 114.7767x; 1.6073x over previous
import functools

import jax
import jax.numpy as jnp
from jax import lax
from jax.experimental import pallas as pl
from jax.experimental.pallas import tpu as pltpu
from jax.experimental.pallas import tpu_sc as plsc

NC = 2
NS = 16
LN = 16

CH = 128
F = 16


def _dot(a, b):
    return jnp.dot(a, b, precision=lax.Precision.HIGHEST)


def _make_edge_kernel(n_nodes, e_pad, n_heads):
    NW = NC * NS
    per_w = e_pad // NW
    assert per_w % (4 * CH) == 0
    n_sub = per_w // CH
    n_acc = ((n_nodes + LN + NS * 8 - 1) // (NS * 8)) * (NS * 8)
    hw = F // n_heads
    rows_per_sub = n_acc // NS

    mesh = plsc.VectorSubcoreMesh(
        core_axis_name="c", subcore_axis_name="s", num_cores=NC, num_subcores=NS
    )

    @functools.partial(
        pl.kernel,
        out_type=(
            jax.ShapeDtypeStruct((NC, n_acc, F), jnp.float32),
            jax.ShapeDtypeStruct((NC, n_acc, F), jnp.float32),
        ),
        mesh=mesh,
        compiler_params=pltpu.CompilerParams(needs_layout_passes=False,
                                             use_tc_tiling_on_sc=False),
        scratch_types=[
            pltpu.MemorySpace.VMEM_SHARED((n_acc, F), jnp.float32),
            pltpu.MemorySpace.VMEM_SHARED((n_acc, F), jnp.float32),
            pltpu.VMEM((4, 2, CH), jnp.int32),
            pltpu.VMEM((2, CH, F), jnp.float32),
            pltpu.VMEM((2, CH, F), jnp.float32),
            pltpu.VMEM((2, CH, F), jnp.float32),
            pltpu.VMEM((2, CH, F), jnp.float32),
            pltpu.VMEM((F,), jnp.float32),
        ] + [pltpu.SemaphoreType.DMA] * 8,
    )
    def edge_kernel(epack, xl_hbm, xr_hbm, att_hbm, zmsg,
                    msg_out, den_out,
                    acc_msg, acc_den, eidx, xl_rows, xr_rows,
                    msg_rows, den_rows, att_v,
                    gl0, gl1, gr0, gr1, scm0, scm1, scd0, scd1):
        cid = lax.axis_index("c")
        sid = lax.axis_index("s")
        wid = sid * NC + cid
        sem_gl = [gl0, gl1]
        sem_gr = [gr0, gr1]
        sem_scm = [scm0, scm1]
        sem_scd = [scd0, scd1]

        @pl.when(sid == 0)
        def _():
            pltpu.sync_copy(zmsg, acc_msg)
            pltpu.sync_copy(zmsg, acc_den)

        pltpu.sync_copy(zmsg.at[pl.ds(0, CH)], den_rows.at[0])
        pltpu.sync_copy(zmsg.at[pl.ds(0, CH)], den_rows.at[1])
        pltpu.sync_copy(att_hbm, att_v)
        plsc.subcore_barrier()

        iota = lax.iota(jnp.int32, LN)
        att_arr = att_v[...]
        base_row = wid * n_sub

        def gather_wait(b, s4):
            pltpu.make_async_copy(xl_hbm.at[eidx.at[s4, 0]], xl_rows.at[b],
                                  sem_gl[b]).wait()
            pltpu.make_async_copy(xr_hbm.at[eidx.at[s4, 1]], xr_rows.at[b],
                                  sem_gr[b]).wait()

        def gather_issue(b, s4):
            pltpu.async_copy(xl_hbm.at[eidx.at[s4, 0]], xl_rows.at[b], sem_gl[b])
            pltpu.async_copy(xr_hbm.at[eidx.at[s4, 1]], xr_rows.at[b], sem_gr[b])

        def scatter_issue(b, s4):
            pltpu.async_copy(msg_rows.at[b], acc_msg.at[eidx.at[s4, 1]],
                             sem_scm[b], add=True)
            pltpu.async_copy(den_rows.at[b], acc_den.at[eidx.at[s4, 1]],
                             sem_scd[b], add=True)

        def scatter_wait(b, s4):
            pltpu.make_async_copy(msg_rows.at[b], acc_msg.at[eidx.at[s4, 1]],
                                  sem_scm[b]).wait()
            pltpu.make_async_copy(den_rows.at[b], acc_den.at[eidx.at[s4, 1]],
                                  sem_scd[b]).wait()

        def compute(b):
            xl_b = xl_rows.at[b]
            xr_b = xr_rows.at[b]
            msg_b = msg_rows.at[b]
            den_b = den_rows.at[b]
            for g in range(CH // LN):
                rows = iota + (g * LN)
                xls = []
                acc = [jnp.zeros((LN,), jnp.float32) for _ in range(n_heads)]
                for c in range(F):
                    colv = jnp.full((LN,), c, jnp.int32)
                    xlc = plsc.load_gather(xl_b, [rows, colv])
                    xrc = plsc.load_gather(xr_b, [rows, colv])
                    m = xlc + xrc
                    lk = jnp.maximum(m, m * 0.2)
                    h_i = c // hw
                    acc[h_i] = acc[h_i] + lk * att_arr[c]
                    xls.append(xlc)
                ws = [jnp.exp(a) for a in acc]
                for c in range(F):
                    colv = jnp.full((LN,), c, jnp.int32)
                    plsc.store_scatter(msg_b, [rows, colv], xls[c] * ws[c // hw])
                for h_i in range(n_heads):
                    colv = jnp.full((LN,), h_i, jnp.int32)
                    plsc.store_scatter(den_b, [rows, colv], ws[h_i])

        pltpu.sync_copy(epack.at[base_row], eidx.at[0])
        gather_issue(0, 0)

        def pipe_body(i, carry):
            for p in range(4):
                b = p % 2
                if p < 2:
                    @pl.when(i > 0)
                    def _():
                        scatter_wait(b, (p + 2) % 4)
                else:
                    scatter_wait(b, (p + 2) % 4)
                pltpu.sync_copy(epack.at[base_row + (4 * i + p) + 1],
                                eidx.at[(p + 1) % 4])
                gather_wait(b, p)
                gather_issue((p + 1) % 2, (p + 1) % 4)
                compute(b)
                scatter_issue(b, p)
            return carry

        lax.fori_loop(0, n_sub // 4, pipe_body, 0)

        gather_wait(0, 0)
        scatter_wait(0, 2)
        scatter_wait(1, 3)
        plsc.subcore_barrier()

        r0 = sid * rows_per_sub
        pltpu.sync_copy(acc_msg.at[pl.ds(r0, rows_per_sub)],
                        msg_out.at[cid, pl.ds(r0, rows_per_sub)])
        pltpu.sync_copy(acc_den.at[pl.ds(r0, rows_per_sub)],
                        den_out.at[cid, pl.ds(r0, rows_per_sub)])

    return edge_kernel


def _tc_pre(x, A_all, blr, attf):
    n = x.shape[0]
    bn = 2000
    grid = n // bn

    def body(x_ref, a_ref, b_ref, att_ref, xl_ref, xr_ref, sm_ref, sd_ref):
        xx = x_ref[...]
        s = xx[:, 5:20]
        xlr = (_dot(xx[:, 0:5], a_ref[0:5, :]) + _dot(s, a_ref[5:20, :])
               + _dot(s * s, a_ref[20:35, :]) + b_ref[...])
        xl = xlr[:, :F]
        xr = xlr[:, F:]
        m = xl + xr
        lk = jnp.maximum(m, m * 0.2) * att_ref[...]
        a0 = jnp.sum(lk[:, :8], axis=1)
        a1 = jnp.sum(lk[:, 8:], axis=1)
        w0 = jnp.exp(a0)[:, None]
        w1 = jnp.exp(a1)[:, None]
        xl_ref[...] = xl
        xr_ref[...] = xr
        sm_ref[...] = jnp.concatenate([xl[:, :8] * w0, xl[:, 8:] * w1], axis=1)
        sd_ref[...] = jnp.concatenate([w0, w1], axis=1)

    full = lambda shape: pl.BlockSpec(shape, lambda i: (0,) * len(shape))
    return pl.pallas_call(
        body,
        grid=(grid,),
        in_specs=[
            pl.BlockSpec((bn, 20), lambda i: (i, 0)),
            full((35, 2 * F)), full((1, 2 * F)), full((1, F)),
        ],
        out_specs=[
            pl.BlockSpec((bn, F), lambda i: (i, 0)),
            pl.BlockSpec((bn, F), lambda i: (i, 0)),
            pl.BlockSpec((bn, F), lambda i: (i, 0)),
            pl.BlockSpec((bn, 2), lambda i: (i, 0)),
        ],
        out_shape=[
            jax.ShapeDtypeStruct((n, F), jnp.float32),
            jax.ShapeDtypeStruct((n, F), jnp.float32),
            jax.ShapeDtypeStruct((n, F), jnp.float32),
            jax.ShapeDtypeStruct((n, 2), jnp.float32),
        ],
    )(x, A_all, blr, attf)


def _tc_mid(mA, mB, dA, dB, sm1, sd1, b1r, W2cat, att2f):
    n = mA.shape[0]
    bn = 2000
    grid = n // bn

    def body(mA_ref, mB_ref, dA_ref, dB_ref, sm_ref, sd_ref, b_ref, w_ref,
             att_ref, xl_ref, xr_ref, sm2_ref, sd2_ref):
        den = dA_ref[...] + dB_ref[...] + sd_ref[...] + 1e-16
        msg = mA_ref[...] + mB_ref[...] + sm_ref[...]
        out1 = jnp.concatenate(
            [msg[:, :8] / den[:, 0:1], msg[:, 8:] / den[:, 1:2]], axis=1)
        v = out1 + b_ref[...]
        h2 = jnp.where(v > 0, v, jnp.exp(v) - 1.0)
        xlr = _dot(h2, w_ref[...])
        xl = xlr[:, :F]
        xr = xlr[:, F:]
        m = xl + xr
        lk = jnp.maximum(m, m * 0.2) * att_ref[...]
        w = jnp.exp(jnp.sum(lk, axis=1))[:, None]
        xl_ref[...] = xl
        xr_ref[...] = xr
        sm2_ref[...] = xl * w
        sd2_ref[...] = w

    full = lambda shape: pl.BlockSpec(shape, lambda i: (0,) * len(shape))
    bspec = lambda w: pl.BlockSpec((bn, w), lambda i: (i, 0))
    return pl.pallas_call(
        body,
        grid=(grid,),
        in_specs=[bspec(F), bspec(F), bspec(2), bspec(2), bspec(F), bspec(2),
                  full((1, F)), full((F, 2 * F)), full((1, F))],
        out_specs=[bspec(F), bspec(F), bspec(F), bspec(1)],
        out_shape=[
            jax.ShapeDtypeStruct((n, F), jnp.float32),
            jax.ShapeDtypeStruct((n, F), jnp.float32),
            jax.ShapeDtypeStruct((n, F), jnp.float32),
            jax.ShapeDtypeStruct((n, 1), jnp.float32),
        ],
    )(mA, mB, dA, dB, sm1, sd1, b1r, W2cat, att2f)


def _tc_final(mA, mB, dA, dB, sm2, sd2, b2r, Wlin, blin):
    n = mA.shape[0]
    bn = 2000
    grid = n // bn

    def body(mA_ref, mB_ref, dA_ref, dB_ref, sm_ref, sd_ref, b_ref, wl_ref,
             bl_ref, y_ref):
        den = dA_ref[...] + dB_ref[...] + sd_ref[...] + 1e-16
        out2 = (mA_ref[...] + mB_ref[...] + sm_ref[...]) / den
        y_ref[...] = _dot(out2 + b_ref[...], wl_ref[...]) + bl_ref[...]

    full = lambda shape: pl.BlockSpec(shape, lambda i: (0,) * len(shape))
    bspec = lambda w: pl.BlockSpec((bn, w), lambda i: (i, 0))
    return pl.pallas_call(
        body,
        grid=(grid,),
        in_specs=[bspec(F), bspec(F), bspec(1), bspec(1), bspec(F), bspec(1),
                  full((1, F)), full((F, 1)), full((1, 1))],
        out_specs=[bspec(1)],
        out_shape=[jax.ShapeDtypeStruct((n, 1), jnp.float32)],
    )(mA, mB, dA, dB, sm2, sd2, b2r, Wlin, blin)


def kernel(x, edge_index, lookup_birth, lookup_gender, symp_tables,
           Wl1, Wr1, att1, b1, Wl2, Wr2, att2, b2, Wlin, blin):
    n = x.shape[0]
    e = edge_index.shape[1]

    t0 = symp_tables[:, 0]
    t1 = symp_tables[:, 1]
    t2 = symp_tables[:, 2]
    lin = (-1.5 * t0 + 2.0 * t1 - 0.5 * t2) / 15.0
    quad = (0.5 * t0 - t1 + 0.5 * t2) / 15.0
    gd = (lookup_gender[1] - lookup_gender[0])[None, :]
    Wh = jnp.concatenate([lookup_birth, gd, lin, quad], axis=0)
    c0 = lookup_gender[0] + t0.sum(axis=0) / 15.0
    Wcat1 = jnp.concatenate([Wl1, Wr1], axis=1)
    A_all = _dot(Wh, Wcat1) / 3.0
    blr = _dot(c0[None, :], Wcat1) / 3.0
    attf1 = att1.reshape(1, F)
    W2cat = jnp.concatenate([Wl2, Wr2], axis=1)
    att2f = att2.reshape(1, F)
    b1r = b1.reshape(1, F)
    b2r = b2.reshape(1, F)
    blinr = blin.reshape(1, 1)

    src = edge_index[0].astype(jnp.int32)
    dst = edge_index[1].astype(jnp.int32)
    group = NC * NS * CH * 4
    e_pad = ((e + group - 1) // group) * group
    npad = e_pad - e
    if npad:
        pad_ids = jnp.arange(npad, dtype=jnp.int32)
        src = jnp.concatenate([src, pad_ids % n])
        dst = jnp.concatenate([dst, n + (pad_ids % LN)])
    src2d = src.reshape(e_pad // CH, CH)
    dst2d = dst.reshape(e_pad // CH, CH)
    epack = jnp.concatenate(
        [jnp.stack([src2d, dst2d], axis=1),
         jnp.zeros((1, 2, CH), jnp.int32)], axis=0)

    n_acc = ((n + LN + NS * 8 - 1) // (NS * 8)) * (NS * 8)
    zmsg = jnp.zeros((n_acc, F), jnp.float32)

    xl1, xr1, sm1, sd1 = _tc_pre(x, A_all, blr, attf1)
    ek1 = _make_edge_kernel(n, e_pad, 2)
    msg_p, den_p = ek1(epack, xl1, xr1, att1.reshape(F), zmsg)

    xl2, xr2, sm2, sd2 = _tc_mid(msg_p[0, :n], msg_p[1, :n],
                                 den_p[0, :n, :2], den_p[1, :n, :2],
                                 sm1, sd1, b1r, W2cat, att2f)
    ek2 = _make_edge_kernel(n, e_pad, 1)
    msg2_p, den2_p = ek2(epack, xl2, xr2, att2.reshape(F), zmsg)

    return _tc_final(msg2_p[0, :n], msg2_p[1, :n],
                     den2_p[0, :n, :1], den2_p[1, :n, :1],
                     sm2, sd2, b2r, Wlin, blinr)[0]


# speedup vs baseline: 121.4629x; 1.0583x over previous
import functools

import jax
import jax.numpy as jnp
from jax import lax
from jax.experimental import pallas as pl
from jax.experimental.pallas import tpu as pltpu
from jax.experimental.pallas import tpu_sc as plsc

NC = 2
NS = 16
LN = 16

CH = 128
F = 16
W = 24


def _dot(a, b):
    return jnp.dot(a, b, precision=lax.Precision.HIGHEST)


def _n_acc(n_nodes):
    return ((n_nodes + LN + NS * 8 - 1) // (NS * 8)) * (NS * 8)


def _make_edge_kernel(n_nodes, e_pad, n_heads):
    NW = NC * NS
    per_w = e_pad // NW
    assert per_w % (4 * CH) == 0
    n_sub = per_w // CH
    n_acc = _n_acc(n_nodes)
    hw = F // n_heads
    rows_per_sub = n_acc // NS

    mesh = plsc.VectorSubcoreMesh(
        core_axis_name="c", subcore_axis_name="s", num_cores=NC, num_subcores=NS
    )

    @functools.partial(
        pl.kernel,
        out_type=jax.ShapeDtypeStruct((NC, n_acc, W), jnp.float32),
        mesh=mesh,
        compiler_params=pltpu.CompilerParams(needs_layout_passes=False,
                                             use_tc_tiling_on_sc=False),
        scratch_types=[
            pltpu.MemorySpace.VMEM_SHARED((n_acc, W), jnp.float32),
            pltpu.VMEM((4, 2, CH), jnp.int32),
            pltpu.VMEM((2, CH, F), jnp.float32),
            pltpu.VMEM((2, CH, F), jnp.float32),
            pltpu.VMEM((2, CH, W), jnp.float32),
            pltpu.VMEM((F,), jnp.float32),
        ] + [pltpu.SemaphoreType.DMA] * 6,
    )
    def edge_kernel(epack, xl_hbm, xr_hbm, att_hbm, zrow, zacc,
                    msg_out,
                    acc, eidx, xl_rows, xr_rows, msg_rows, att_v,
                    gl0, gl1, gr0, gr1, sc0, sc1):
        cid = lax.axis_index("c")
        sid = lax.axis_index("s")
        wid = sid * NC + cid
        sem_gl = [gl0, gl1]
        sem_gr = [gr0, gr1]
        sem_sc = [sc0, sc1]

        @pl.when(sid == 0)
        def _():
            pltpu.sync_copy(zacc, acc)

        pltpu.sync_copy(zrow, msg_rows.at[0])
        pltpu.sync_copy(zrow, msg_rows.at[1])
        pltpu.sync_copy(att_hbm, att_v)
        plsc.subcore_barrier()

        iota = lax.iota(jnp.int32, LN)
        att_arr = att_v[...]
        base_row = wid * n_sub

        def gather_issue(b, s4):
            pltpu.async_copy(xl_hbm.at[eidx.at[s4, 0]], xl_rows.at[b], sem_gl[b])
            pltpu.async_copy(xr_hbm.at[eidx.at[s4, 1]], xr_rows.at[b], sem_gr[b])

        def gather_wait(b, s4):
            pltpu.make_async_copy(xl_hbm.at[eidx.at[s4, 0]], xl_rows.at[b],
                                  sem_gl[b]).wait()
            pltpu.make_async_copy(xr_hbm.at[eidx.at[s4, 1]], xr_rows.at[b],
                                  sem_gr[b]).wait()

        def scatter_issue(b, s4):
            pltpu.async_copy(msg_rows.at[b], acc.at[eidx.at[s4, 1]],
                             sem_sc[b], add=True)

        def scatter_wait(b, s4):
            pltpu.make_async_copy(msg_rows.at[b], acc.at[eidx.at[s4, 1]],
                                  sem_sc[b]).wait()

        def compute(b):
            xl_b = xl_rows.at[b]
            xr_b = xr_rows.at[b]
            msg_b = msg_rows.at[b]
            for g in range(CH // LN):
                rows = iota + (g * LN)
                xls = []
                acc_a = [jnp.zeros((LN,), jnp.float32) for _ in range(n_heads)]
                for c in range(F):
                    colv = jnp.full((LN,), c, jnp.int32)
                    xlc = plsc.load_gather(xl_b, [rows, colv])
                    xrc = plsc.load_gather(xr_b, [rows, colv])
                    m = xlc + xrc
                    lk = jnp.maximum(m, m * 0.2)
                    h_i = c // hw
                    acc_a[h_i] = acc_a[h_i] + lk * att_arr[c]
                    xls.append(xlc)
                ws = [jnp.exp(a) for a in acc_a]
                for c in range(F):
                    colv = jnp.full((LN,), c, jnp.int32)
                    plsc.store_scatter(msg_b, [rows, colv], xls[c] * ws[c // hw])
                for h_i in range(n_heads):
                    colv = jnp.full((LN,), F + h_i, jnp.int32)
                    plsc.store_scatter(msg_b, [rows, colv], ws[h_i])

        pltpu.sync_copy(epack.at[base_row], eidx.at[0])
        gather_issue(0, 0)

        def pipe_body(i, carry):
            for p in range(4):
                b = p % 2
                j = 4 * i + p
                if p < 2:
                    @pl.when(i > 0)
                    def _():
                        scatter_wait(b, (p + 2) % 4)
                else:
                    scatter_wait(b, (p + 2) % 4)
                pltpu.sync_copy(epack.at[base_row + j + 1],
                                eidx.at[(p + 1) % 4])
                gather_wait(b, p)
                gather_issue((p + 1) % 2, (p + 1) % 4)
                compute(b)
                scatter_issue(b, p)
            return carry

        lax.fori_loop(0, n_sub // 4, pipe_body, 0)

        gather_wait(0, 0)
        scatter_wait(0, 2)
        scatter_wait(1, 3)
        plsc.subcore_barrier()

        r0 = sid * rows_per_sub
        pltpu.sync_copy(acc.at[pl.ds(r0, rows_per_sub)],
                        msg_out.at[cid, pl.ds(r0, rows_per_sub)])

    return edge_kernel


def _tc_pre(x, A_all, blr, attf):
    n = x.shape[0]
    bn = 3136
    grid = n // bn

    def body(x_ref, a_ref, b_ref, att_ref, xl_ref, xr_ref, sm_ref, sd_ref):
        xx = x_ref[...]
        s = xx[:, 5:20]
        xlr = (_dot(xx[:, 0:5], a_ref[0:5, :]) + _dot(s, a_ref[5:20, :])
               + _dot(s * s, a_ref[20:35, :]) + b_ref[...])
        xl = xlr[:, :F]
        xr = xlr[:, F:]
        m = xl + xr
        lk = jnp.maximum(m, m * 0.2) * att_ref[...]
        a0 = jnp.sum(lk[:, :8], axis=1)
        a1 = jnp.sum(lk[:, 8:], axis=1)
        w0 = jnp.exp(a0)[:, None]
        w1 = jnp.exp(a1)[:, None]
        xl_ref[...] = xl
        xr_ref[...] = xr
        sm_ref[...] = jnp.concatenate([xl[:, :8] * w0, xl[:, 8:] * w1], axis=1)
        sd_ref[...] = jnp.concatenate([w0, w1], axis=1)

    full = lambda shape: pl.BlockSpec(shape, lambda i: (0,) * len(shape))
    bspec = lambda w: pl.BlockSpec((bn, w), lambda i: (i, 0))
    return pl.pallas_call(
        body,
        grid=(grid,),
        in_specs=[
            bspec(20),
            full((35, 2 * F)), full((1, 2 * F)), full((1, F)),
        ],
        out_specs=[bspec(F), bspec(F), bspec(F), bspec(2)],
        out_shape=[
            jax.ShapeDtypeStruct((n, F), jnp.float32),
            jax.ShapeDtypeStruct((n, F), jnp.float32),
            jax.ShapeDtypeStruct((n, F), jnp.float32),
            jax.ShapeDtypeStruct((n, 2), jnp.float32),
        ],
    )(x, A_all, blr, attf)


def _tc_mid(msg_p, sm1, sd1, b1r, W2cat, att2f):
    n = sm1.shape[0]
    bn = 3136
    grid = n // bn

    def body(mp_ref, sm_ref, sd_ref, b_ref, w_ref,
             att_ref, xl_ref, xr_ref, sm2_ref, sd2_ref):
        mp = mp_ref[0] + mp_ref[1]
        den = mp[:, F:F + 2] + sd_ref[...] + 1e-16
        msg = mp[:, :F] + sm_ref[...]
        out1 = jnp.concatenate(
            [msg[:, :8] / den[:, 0:1], msg[:, 8:] / den[:, 1:2]], axis=1)
        v = out1 + b_ref[...]
        h2 = jnp.where(v > 0, v, jnp.exp(v) - 1.0)
        xlr = _dot(h2, w_ref[...])
        xl = xlr[:, :F]
        xr = xlr[:, F:]
        m = xl + xr
        lk = jnp.maximum(m, m * 0.2) * att_ref[...]
        w = jnp.exp(jnp.sum(lk, axis=1))[:, None]
        xl_ref[...] = xl
        xr_ref[...] = xr
        sm2_ref[...] = xl * w
        sd2_ref[...] = w

    full = lambda shape: pl.BlockSpec(shape, lambda i: (0,) * len(shape))
    bspec = lambda w: pl.BlockSpec((bn, w), lambda i: (i, 0))
    return pl.pallas_call(
        body,
        grid=(grid,),
        in_specs=[pl.BlockSpec((NC, bn, W), lambda i: (0, i, 0)),
                  bspec(F), bspec(2),
                  full((1, F)), full((F, 2 * F)), full((1, F))],
        out_specs=[bspec(F), bspec(F), bspec(F), bspec(1)],
        out_shape=[
            jax.ShapeDtypeStruct((n, F), jnp.float32),
            jax.ShapeDtypeStruct((n, F), jnp.float32),
            jax.ShapeDtypeStruct((n, F), jnp.float32),
            jax.ShapeDtypeStruct((n, 1), jnp.float32),
        ],
    )(msg_p, sm1, sd1, b1r, W2cat, att2f)


def _tc_final(msg_p, sm2, sd2, b2r, Wlin, blin):
    n = sm2.shape[0]
    bn = 3136
    grid = n // bn

    def body(mp_ref, sm_ref, sd_ref, b_ref, wl_ref, bl_ref, y_ref):
        mp = mp_ref[0] + mp_ref[1]
        den = mp[:, F:F + 1] + sd_ref[...] + 1e-16
        out2 = (mp[:, :F] + sm_ref[...]) / den
        y_ref[...] = _dot(out2 + b_ref[...], wl_ref[...]) + bl_ref[...]

    full = lambda shape: pl.BlockSpec(shape, lambda i: (0,) * len(shape))
    bspec = lambda w: pl.BlockSpec((bn, w), lambda i: (i, 0))
    return pl.pallas_call(
        body,
        grid=(grid,),
        in_specs=[pl.BlockSpec((NC, bn, W), lambda i: (0, i, 0)),
                  bspec(F), bspec(1),
                  full((1, F)), full((F, 1)), full((1, 1))],
        out_specs=[bspec(1)],
        out_shape=[jax.ShapeDtypeStruct((n, 1), jnp.float32)],
    )(msg_p, sm2, sd2, b2r, Wlin, blin)


def kernel(x, edge_index, lookup_birth, lookup_gender, symp_tables,
           Wl1, Wr1, att1, b1, Wl2, Wr2, att2, b2, Wlin, blin):
    n = x.shape[0]
    e = edge_index.shape[1]
    n_acc = _n_acc(n)

    t0 = symp_tables[:, 0]
    t1 = symp_tables[:, 1]
    t2 = symp_tables[:, 2]
    lin = (-1.5 * t0 + 2.0 * t1 - 0.5 * t2) / 15.0
    quad = (0.5 * t0 - t1 + 0.5 * t2) / 15.0
    gd = (lookup_gender[1] - lookup_gender[0])[None, :]
    Wh = jnp.concatenate([lookup_birth, gd, lin, quad], axis=0)
    c0 = lookup_gender[0] + t0.sum(axis=0) / 15.0
    Wcat1 = jnp.concatenate([Wl1, Wr1], axis=1)
    A_all = _dot(Wh, Wcat1) / 3.0
    blr = _dot(c0[None, :], Wcat1) / 3.0
    attf1 = att1.reshape(1, F)
    W2cat = jnp.concatenate([Wl2, Wr2], axis=1)
    att2f = att2.reshape(1, F)
    b1r = b1.reshape(1, F)
    b2r = b2.reshape(1, F)
    blinr = blin.reshape(1, 1)

    src = edge_index[0].astype(jnp.int32)
    dst = edge_index[1].astype(jnp.int32)
    group = NC * NS * CH * 4
    e_pad = ((e + group - 1) // group) * group
    npad = e_pad - e
    if npad:
        pad_ids = jnp.arange(npad, dtype=jnp.int32)
        src = jnp.concatenate([src, pad_ids % n])
        dst = jnp.concatenate([dst, n + (pad_ids % LN)])
    src2d = src.reshape(e_pad // CH, CH)
    dst2d = dst.reshape(e_pad // CH, CH)
    epack = jnp.concatenate(
        [jnp.stack([src2d, dst2d], axis=1),
         jnp.zeros((2, 2, CH), jnp.int32)], axis=0)

    zrow = jnp.zeros((CH, W), jnp.float32)
    zacc = jnp.zeros((n_acc, W), jnp.float32)
    x_pad = jnp.pad(x, ((0, n_acc - n), (0, 0)))

    xl1, xr1, sm1, sd1 = _tc_pre(x_pad, A_all, blr, attf1)
    ek1 = _make_edge_kernel(n, e_pad, 2)
    msg_p = ek1(epack, xl1, xr1, att1.reshape(F), zrow, zacc)

    xl2, xr2, sm2, sd2 = _tc_mid(msg_p, sm1, sd1, b1r, W2cat, att2f)
    ek2 = _make_edge_kernel(n, e_pad, 1)
    msg2_p = ek2(epack, xl2, xr2, att2.reshape(F), zrow, zacc)

    y = _tc_final(msg2_p, sm2, sd2, b2r, Wlin, blinr)[0]
    return y[:n]
